# baseline (speedup 1.0000x reference)
import functools

import jax
import jax.numpy as jnp
from jax import lax
from jax.experimental import pallas as pl
from jax.experimental.pallas import tpu as pltpu
from jax.experimental.pallas import tpu_sc as plsc

N = 10000
NP = 10240
E = 320000
EP = 327680
NC, NS = 2, 16
EPC = EP // NC
BLK = 4096
NB = EPC // BLK
CPT = 4
NCHUNK = 80
CL = 128
RPT = NP // NS
IN_CH, HID, OUT_CH, NG = 128, 64, 128, 16

_MESH = plsc.VectorSubcoreMesh(
    core_axis_name="c", subcore_axis_name="s", num_cores=NC, num_subcores=NS)
_SC_PARAMS = pltpu.CompilerParams(
    use_tc_tiling_on_sc=False, needs_layout_passes=False)


@functools.partial(
    pl.kernel,
    out_type=jax.ShapeDtypeStruct((NC, NP), jnp.float32),
    mesh=_MESH,
    compiler_params=_SC_PARAMS,
    scratch_types=[
        pltpu.VMEM((NCHUNK, CL), jnp.int32),
        pltpu.VMEM((RPT,), jnp.float32),
        pltpu.VMEM((CL,), jnp.float32),
        pltpu.VMEM_SHARED((NP,), jnp.float32),
    ],
)
def _deg_kernel(dst_hbm, out_hbm, dst_v, zb, ones_v, acc):
    c = lax.axis_index("c")
    s = lax.axis_index("s")
    w = c * NS + s

    def zb_body(i, _):
        zb[pl.ds(i * 16, 16)] = jnp.zeros((16,), jnp.float32)
        return 0
    lax.fori_loop(0, RPT // 16, zb_body, 0)

    def ones_body(i, _):
        ones_v[pl.ds(i * 16, 16)] = jnp.ones((16,), jnp.float32)
        return 0
    lax.fori_loop(0, CL // 16, ones_body, 0)

    pltpu.sync_copy(zb, acc.at[pl.ds(s * RPT, RPT)])
    pltpu.sync_copy(dst_hbm.at[w], dst_v)
    plsc.subcore_barrier()

    def body(j, _):
        pltpu.sync_copy(ones_v, acc.at[dst_v.at[j]], add=True)
        return 0
    lax.fori_loop(0, NCHUNK, body, 0)

    plsc.subcore_barrier()
    pltpu.sync_copy(acc.at[pl.ds(s * RPT, RPT)],
                    out_hbm.at[c, pl.ds(s * RPT, RPT)])


@functools.partial(
    pl.kernel,
    out_type=jax.ShapeDtypeStruct((NC, HID, NP), jnp.float32),
    mesh=_MESH,
    compiler_params=_SC_PARAMS,
    scratch_types=[
        pltpu.VMEM((CPT, NP), jnp.float32),
        pltpu.VMEM((CPT, NP), jnp.float32),
        pltpu.VMEM((BLK,), jnp.int32),
        pltpu.VMEM((BLK,), jnp.int32),
        pltpu.VMEM((BLK,), jnp.int32),
        pltpu.VMEM((BLK,), jnp.int32),
        pltpu.SemaphoreType.DMA,
        pltpu.SemaphoreType.DMA,
    ],
)
def _aggT(table_hbm, src_hbm, dst_hbm, out_hbm, tab_v, acc_v,
          sbufa, dbufa, sbufb, dbufb, sema, semb):
    c = lax.axis_index("c")
    s = lax.axis_index("s")

    pltpu.sync_copy(table_hbm.at[pl.ds(s * CPT, CPT)], tab_v)
    pltpu.sync_copy(table_hbm.at[pl.ds(s * CPT, CPT)], acc_v)

    def start_load(b, sb, db, sem):
        pltpu.async_copy(src_hbm.at[c, b], sb, sem)
        pltpu.async_copy(dst_hbm.at[c, b], db, sem)

    def wait_load(sb, db, sem):
        pltpu.make_async_copy(src_hbm.at[c, 0], sb, sem).wait()
        pltpu.make_async_copy(src_hbm.at[c, 0], db, sem).wait()

    def compute(sb, db):
        @plsc.parallel_loop(0, BLK, step=16, unroll=8)
        def _(i):
            s16 = sb[pl.ds(i, 16)]
            d16 = db[pl.ds(i, 16)]
            for cc in range(CPT):
                cv = jnp.full((16,), cc, jnp.int32)
                v = plsc.load_gather(tab_v, [cv, s16])
                plsc.addupdate_scatter(acc_v, [cv, d16], v)

    start_load(0, sbufa, dbufa, sema)

    def body(k, _):
        b = 2 * k
        wait_load(sbufa, dbufa, sema)
        start_load(b + 1, sbufb, dbufb, semb)
        compute(sbufa, dbufa)
        wait_load(sbufb, dbufb, semb)

        @pl.when(k < NB // 2 - 1)
        def _():
            start_load(b + 2, sbufa, dbufa, sema)

        compute(sbufb, dbufb)
        return 0
    lax.fori_loop(0, NB // 2, body, 0)

    pltpu.sync_copy(acc_v, out_hbm.at[c, pl.ds(s * CPT, CPT)])


def _pre_body(degp_ref, xt_ref, wt_ref, h_ref, dis_ref):
    deg = degp_ref[0] + degp_ref[1] + 1.0
    dis = lax.rsqrt(deg)
    dis_ref[...] = dis
    h = jnp.dot(wt_ref[...], xt_ref[...], preferred_element_type=jnp.float32)
    h_ref[...] = h * dis


def _bn_relu(z, g_ref, be_ref):
    cols = lax.broadcasted_iota(jnp.int32, z.shape, 1)
    mask = cols < N
    zm = jnp.where(mask, z, 0.0)
    mean = jnp.sum(zm, axis=1, keepdims=True) / N
    var = jnp.sum(zm * zm, axis=1, keepdims=True) / N - mean * mean
    y = g_ref[...] * (z - mean) * lax.rsqrt(var + 1e-5) + be_ref[...]
    return jnp.where(mask, jnp.maximum(y, 0.0), 0.0)


def _mid_body(sp_ref, hp_ref, dis_ref, b_ref, g_ref, be_ref, wt_ref, out_ref):
    dis = dis_ref[...]
    z = dis * (sp_ref[0] + sp_ref[1] - hp_ref[...]) + b_ref[...]
    y = _bn_relu(z, g_ref, be_ref)
    out_ref[...] = jnp.dot(
        wt_ref[...], y, preferred_element_type=jnp.float32) * dis


def _mid_split_body(sp_ref, hp_ref, dis_ref, b_ref, g_ref, be_ref, wt_ref,
                    outa_ref, outb_ref):
    dis = dis_ref[...]
    z = dis * (sp_ref[0] + sp_ref[1] - hp_ref[...]) + b_ref[...]
    y = _bn_relu(z, g_ref, be_ref)
    outa_ref[...] = jnp.dot(
        wt_ref[0:HID, :], y, preferred_element_type=jnp.float32) * dis
    outb_ref[...] = jnp.dot(
        wt_ref[HID:OUT_CH, :], y, preferred_element_type=jnp.float32) * dis


def _final_body(spa_ref, spb_ref, hpa_ref, hpb_ref, dis_ref, b_ref,
                batch_ref, out_ref):
    dis = dis_ref[...]
    za = dis * (spa_ref[0] + spa_ref[1] - hpa_ref[...]) + b_ref[0:HID, :]
    zb = dis * (spb_ref[0] + spb_ref[1] - hpb_ref[...]) + b_ref[HID:OUT_CH, :]
    gid = lax.broadcasted_iota(jnp.int32, (NP, NG), 1)
    onehot = (batch_ref[...] == gid).astype(jnp.float32)
    cnt = jnp.maximum(jnp.sum(onehot, axis=0, keepdims=True), 1.0)
    pa = jnp.dot(za, onehot, preferred_element_type=jnp.float32) / cnt
    pb = jnp.dot(zb, onehot, preferred_element_type=jnp.float32) / cnt
    out_ref[0:HID, :] = 1.0 / (1.0 + jnp.exp(-pa))
    out_ref[HID:OUT_CH, :] = 1.0 / (1.0 + jnp.exp(-pb))


def _f32(*shape):
    return jax.ShapeDtypeStruct(shape, jnp.float32)


def kernel(x, edge_index, batch, W0, b0, W1, b1, W2, b2, g0, be0, g1, be1):
    src = edge_index[0].astype(jnp.int32)
    dst = edge_index[1].astype(jnp.int32)
    pad = jnp.full((EP - E,), N, jnp.int32)
    src_p = jnp.concatenate([src, pad])
    dst_p = jnp.concatenate([dst, pad])
    src_r = src_p.reshape(NC, NB, BLK)
    dst_r = dst_p.reshape(NC, NB, BLK)
    dst_deg = dst_p.reshape(NC * NS, NCHUNK, CL)
    xt_p = jnp.pad(x, ((0, NP - N), (0, 0))).T
    batch_p = jnp.concatenate(
        [batch.astype(jnp.int32),
         jnp.full((NP - N,), NG, jnp.int32)]).reshape(NP, 1)

    degp = _deg_kernel(dst_deg).reshape(NC, 1, NP)

    h0, dis = pl.pallas_call(
        _pre_body, out_shape=(_f32(HID, NP), _f32(1, NP)))(degp, xt_p, W0.T)

    s0 = _aggT(h0, src_r, dst_r)
    h1 = pl.pallas_call(_mid_body, out_shape=_f32(HID, NP))(
        s0, h0, dis, b0.reshape(-1, 1), g0.reshape(-1, 1),
        be0.reshape(-1, 1), W1.T)

    s1 = _aggT(h1, src_r, dst_r)
    h2a, h2b = pl.pallas_call(
        _mid_split_body, out_shape=(_f32(HID, NP), _f32(HID, NP)))(
            s1, h1, dis, b1.reshape(-1, 1), g1.reshape(-1, 1),
            be1.reshape(-1, 1), W2.T)

    s2a = _aggT(h2a, src_r, dst_r)
    s2b = _aggT(h2b, src_r, dst_r)
    outT = pl.pallas_call(_final_body, out_shape=_f32(OUT_CH, NG))(
        s2a, s2b, h2a, h2b, dis, b2.reshape(-1, 1), batch_p)
    return outT.T


# speedup vs baseline: 1.0975x; 1.0975x over previous
import functools

import jax
import jax.numpy as jnp
from jax import lax
from jax.experimental import pallas as pl
from jax.experimental.pallas import tpu as pltpu
from jax.experimental.pallas import tpu_sc as plsc

N = 10000
NP = 10240
E = 320000
EP = 327680
NC, NS = 2, 16
EPC = EP // NC
BLK = 4096
NB = EPC // BLK
CPT = 4
NCHUNK = 80
CL = 128
RPT = NP // NS
IN_CH, HID, OUT_CH, NG = 128, 64, 128, 16

_MESH = plsc.VectorSubcoreMesh(
    core_axis_name="c", subcore_axis_name="s", num_cores=NC, num_subcores=NS)
_SC_PARAMS = pltpu.CompilerParams(
    use_tc_tiling_on_sc=False, needs_layout_passes=False)


@functools.partial(
    pl.kernel,
    out_type=jax.ShapeDtypeStruct((NC, NP), jnp.float32),
    mesh=_MESH,
    compiler_params=_SC_PARAMS,
    scratch_types=[
        pltpu.VMEM((NCHUNK, CL), jnp.int32),
        pltpu.VMEM((RPT,), jnp.float32),
        pltpu.VMEM((CL,), jnp.float32),
        pltpu.VMEM_SHARED((NP,), jnp.float32),
    ],
)
def _deg_kernel(dst_hbm, out_hbm, dst_v, zb, ones_v, acc):
    c = lax.axis_index("c")
    s = lax.axis_index("s")
    w = c * NS + s

    def zb_body(i, _):
        zb[pl.ds(i * 16, 16)] = jnp.zeros((16,), jnp.float32)
        return 0
    lax.fori_loop(0, RPT // 16, zb_body, 0)

    def ones_body(i, _):
        ones_v[pl.ds(i * 16, 16)] = jnp.ones((16,), jnp.float32)
        return 0
    lax.fori_loop(0, CL // 16, ones_body, 0)

    pltpu.sync_copy(zb, acc.at[pl.ds(s * RPT, RPT)])
    pltpu.sync_copy(dst_hbm.at[w], dst_v)
    plsc.subcore_barrier()

    def body(j, _):
        pltpu.sync_copy(ones_v, acc.at[dst_v.at[j]], add=True)
        return 0
    lax.fori_loop(0, NCHUNK, body, 0)

    plsc.subcore_barrier()
    pltpu.sync_copy(acc.at[pl.ds(s * RPT, RPT)],
                    out_hbm.at[c, pl.ds(s * RPT, RPT)])


@functools.partial(
    pl.kernel,
    out_type=jax.ShapeDtypeStruct((NC, HID, NP), jnp.float32),
    mesh=_MESH,
    compiler_params=_SC_PARAMS,
    scratch_types=[
        pltpu.VMEM((CPT, NP), jnp.float32),
        pltpu.VMEM((CPT, NP), jnp.float32),
        pltpu.VMEM((BLK,), jnp.int32),
        pltpu.VMEM((BLK,), jnp.int32),
        pltpu.VMEM((BLK,), jnp.int32),
        pltpu.VMEM((BLK,), jnp.int32),
        pltpu.SemaphoreType.DMA,
        pltpu.SemaphoreType.DMA,
    ],
)
def _aggT(table_hbm, src_hbm, dst_hbm, out_hbm, tab_v, acc_v,
          sbufa, dbufa, sbufb, dbufb, sema, semb):
    c = lax.axis_index("c")
    s = lax.axis_index("s")

    pltpu.sync_copy(table_hbm.at[pl.ds(s * CPT, CPT)], tab_v)
    pltpu.sync_copy(table_hbm.at[pl.ds(s * CPT, CPT)], acc_v)

    def start_load(b, sb, db, sem):
        pltpu.async_copy(src_hbm.at[c, b], sb, sem)
        pltpu.async_copy(dst_hbm.at[c, b], db, sem)

    def wait_load(sb, db, sem):
        pltpu.make_async_copy(src_hbm.at[c, 0], sb, sem).wait()
        pltpu.make_async_copy(src_hbm.at[c, 0], db, sem).wait()

    def compute(sb, db):
        @plsc.parallel_loop(0, BLK, step=16, unroll=4)
        def _(i):
            s16 = sb[pl.ds(i, 16)]
            d16 = db[pl.ds(i, 16)]
            for cc in range(CPT):
                cv = jnp.full((16,), cc, jnp.int32)
                v = plsc.load_gather(tab_v, [cv, s16])
                plsc.addupdate_scatter(acc_v, [cv, d16], v)

    start_load(0, sbufa, dbufa, sema)

    def body(k, _):
        b = 2 * k
        wait_load(sbufa, dbufa, sema)
        start_load(b + 1, sbufb, dbufb, semb)
        compute(sbufa, dbufa)
        wait_load(sbufb, dbufb, semb)

        @pl.when(k < NB // 2 - 1)
        def _():
            start_load(b + 2, sbufa, dbufa, sema)

        compute(sbufb, dbufb)
        return 0
    lax.fori_loop(0, NB // 2, body, 0)

    pltpu.sync_copy(acc_v, out_hbm.at[c, pl.ds(s * CPT, CPT)])


@functools.partial(
    pl.kernel,
    out_type=jax.ShapeDtypeStruct((OUT_CH, NP), jnp.float32),
    mesh=_MESH,
    compiler_params=_SC_PARAMS,
    scratch_types=[
        pltpu.VMEM((CPT, NP), jnp.float32),
        pltpu.VMEM((CPT, NP), jnp.float32),
        pltpu.VMEM((BLK,), jnp.int32),
        pltpu.VMEM((BLK,), jnp.int32),
        pltpu.VMEM((BLK,), jnp.int32),
        pltpu.VMEM((BLK,), jnp.int32),
        pltpu.SemaphoreType.DMA,
        pltpu.SemaphoreType.DMA,
    ],
)
def _aggT_wide(table_hbm, src_hbm, dst_hbm, out_hbm, tab_v, acc_v,
               sbufa, dbufa, sbufb, dbufb, sema, semb):
    c = lax.axis_index("c")
    s = lax.axis_index("s")
    row = c * (OUT_CH // NC) + s * CPT

    pltpu.sync_copy(table_hbm.at[pl.ds(row, CPT)], tab_v)
    pltpu.sync_copy(table_hbm.at[pl.ds(row, CPT)], acc_v)

    def start_load(b, sb, db, sem):
        pltpu.async_copy(src_hbm.at[b], sb, sem)
        pltpu.async_copy(dst_hbm.at[b], db, sem)

    def wait_load(sb, db, sem):
        pltpu.make_async_copy(src_hbm.at[0], sb, sem).wait()
        pltpu.make_async_copy(src_hbm.at[0], db, sem).wait()

    def compute(sb, db):
        @plsc.parallel_loop(0, BLK, step=16, unroll=4)
        def _(i):
            s16 = sb[pl.ds(i, 16)]
            d16 = db[pl.ds(i, 16)]
            for cc in range(CPT):
                cv = jnp.full((16,), cc, jnp.int32)
                v = plsc.load_gather(tab_v, [cv, s16])
                plsc.addupdate_scatter(acc_v, [cv, d16], v)

    NBW = EP // BLK
    start_load(0, sbufa, dbufa, sema)

    def body(k, _):
        b = 2 * k
        wait_load(sbufa, dbufa, sema)
        start_load(b + 1, sbufb, dbufb, semb)
        compute(sbufa, dbufa)
        wait_load(sbufb, dbufb, semb)

        @pl.when(k < NBW // 2 - 1)
        def _():
            start_load(b + 2, sbufa, dbufa, sema)

        compute(sbufb, dbufb)
        return 0
    lax.fori_loop(0, NBW // 2, body, 0)

    pltpu.sync_copy(acc_v, out_hbm.at[pl.ds(row, CPT)])


def _pre_body(degp_ref, xt_ref, wt_ref, h_ref, dis_ref):
    deg = degp_ref[0] + degp_ref[1] + 1.0
    dis = lax.rsqrt(deg)
    dis_ref[...] = dis
    h = jnp.dot(wt_ref[...], xt_ref[...], preferred_element_type=jnp.float32)
    h_ref[...] = h * dis


def _bn_relu(z, g_ref, be_ref):
    cols = lax.broadcasted_iota(jnp.int32, z.shape, 1)
    mask = cols < N
    zm = jnp.where(mask, z, 0.0)
    mean = jnp.sum(zm, axis=1, keepdims=True) / N
    var = jnp.sum(zm * zm, axis=1, keepdims=True) / N - mean * mean
    y = g_ref[...] * (z - mean) * lax.rsqrt(var + 1e-5) + be_ref[...]
    return jnp.where(mask, jnp.maximum(y, 0.0), 0.0)


def _mid_body(sp_ref, hp_ref, dis_ref, b_ref, g_ref, be_ref, wt_ref, out_ref):
    dis = dis_ref[...]
    z = dis * (sp_ref[0] + sp_ref[1] - hp_ref[...]) + b_ref[...]
    y = _bn_relu(z, g_ref, be_ref)
    out_ref[...] = jnp.dot(
        wt_ref[...], y, preferred_element_type=jnp.float32) * dis


def _final_body(s2_ref, dis_ref, b_ref, batch_ref, out_ref):
    z = dis_ref[...] * s2_ref[...] + b_ref[...]
    gid = lax.broadcasted_iota(jnp.int32, (NP, NG), 1)
    onehot = (batch_ref[...] == gid).astype(jnp.float32)
    cnt = jnp.maximum(jnp.sum(onehot, axis=0, keepdims=True), 1.0)
    p = jnp.dot(z, onehot, preferred_element_type=jnp.float32) / cnt
    out_ref[...] = 1.0 / (1.0 + jnp.exp(-p))


def _f32(*shape):
    return jax.ShapeDtypeStruct(shape, jnp.float32)


def kernel(x, edge_index, batch, W0, b0, W1, b1, W2, b2, g0, be0, g1, be1):
    src = edge_index[0].astype(jnp.int32)
    dst = edge_index[1].astype(jnp.int32)
    pad = jnp.full((EP - E,), N, jnp.int32)
    src_p = jnp.concatenate([src, pad])
    dst_p = jnp.concatenate([dst, pad])
    src_r = src_p.reshape(NC, NB, BLK)
    dst_r = dst_p.reshape(NC, NB, BLK)
    src_w = src_p.reshape(EP // BLK, BLK)
    dst_w = dst_p.reshape(EP // BLK, BLK)
    dst_deg = dst_p.reshape(NC * NS, NCHUNK, CL)
    xt_p = jnp.pad(x, ((0, NP - N), (0, 0))).T
    batch_p = jnp.concatenate(
        [batch.astype(jnp.int32),
         jnp.full((NP - N,), NG, jnp.int32)]).reshape(NP, 1)

    degp = _deg_kernel(dst_deg).reshape(NC, 1, NP)

    h0, dis = pl.pallas_call(
        _pre_body, out_shape=(_f32(HID, NP), _f32(1, NP)))(degp, xt_p, W0.T)

    s0 = _aggT(h0, src_r, dst_r)
    h1 = pl.pallas_call(_mid_body, out_shape=_f32(HID, NP))(
        s0, h0, dis, b0.reshape(-1, 1), g0.reshape(-1, 1),
        be0.reshape(-1, 1), W1.T)

    s1 = _aggT(h1, src_r, dst_r)
    h2 = pl.pallas_call(_mid_body, out_shape=_f32(OUT_CH, NP))(
        s1, h1, dis, b1.reshape(-1, 1), g1.reshape(-1, 1),
        be1.reshape(-1, 1), W2.T)

    s2 = _aggT_wide(h2, src_w, dst_w)
    outT = pl.pallas_call(_final_body, out_shape=_f32(OUT_CH, NG))(
        s2, dis, b2.reshape(-1, 1), batch_p)
    return outT.T
